# baseline (speedup 1.0000x reference)
import functools
import jax
import jax.numpy as jnp
from jax.experimental import pallas as pl
from jax.experimental.pallas import tpu as pltpu

N = 50000
HID = 128
ROW_BLK = 1000


def _mm_kernel(x_ref, w_ref, b_ref, o_ref, *, relu):
    acc = jnp.dot(x_ref[...], w_ref[...], preferred_element_type=jnp.float32)
    acc = acc + b_ref[...]
    if relu:
        acc = jnp.maximum(acc, 0.0)
    o_ref[...] = acc


def _mm(x, w, b, relu=True):
    n, k = x.shape
    m = w.shape[1]
    grid = (n // ROW_BLK,)
    return pl.pallas_call(
        functools.partial(_mm_kernel, relu=relu),
        grid=grid,
        in_specs=[
            pl.BlockSpec((ROW_BLK, k), lambda i: (i, 0)),
            pl.BlockSpec((k, m), lambda i: (0, 0)),
            pl.BlockSpec((1, m), lambda i: (0, 0)),
        ],
        out_specs=pl.BlockSpec((ROW_BLK, m), lambda i: (i, 0)),
        out_shape=jax.ShapeDtypeStruct((n, m), jnp.float32),
    )(x, w, b)


def _gat(h_in, src, dst, W, a_src, a_dst, bias, heads, out_ch, concat):
    n = h_in.shape[0]
    h = _mm(h_in, W, jnp.zeros((1, W.shape[1]), jnp.float32), relu=False)
    h = h.reshape(n, heads, out_ch)
    al_s = jnp.sum(h * a_src[None, :, :], axis=-1)
    al_d = jnp.sum(h * a_dst[None, :, :], axis=-1)
    e = al_s[src] + al_d[dst]
    e = jax.nn.leaky_relu(e, 0.2)
    m = jax.ops.segment_max(e, dst, num_segments=n)
    m = jnp.where(jnp.isfinite(m), m, 0.0)
    ex = jnp.exp(e - m[dst])
    den = jax.ops.segment_sum(ex, dst, num_segments=n)
    alpha = ex / (den[dst] + 1e-16)
    msg = h[src] * alpha[:, :, None]
    agg = jax.ops.segment_sum(msg, dst, num_segments=n)
    if concat:
        out = agg.reshape(n, heads * out_ch)
    else:
        out = jnp.mean(agg, axis=1)
    return out + bias


def _bn(x, g, b, m, v):
    return (x - m) / jnp.sqrt(v + 1e-5) * g + b


def kernel(x, edge_index, platform_x, params):
    p = params
    src = edge_index[0]
    dst = edge_index[1]
    pad = (-N) % ROW_BLK
    xp = jnp.pad(x, ((0, pad), (0, 0))) if pad else x
    h = _mm(xp, p["ne_W"], p["ne_b"][None, :], relu=True)[:N]
    h = _gat(h, src, dst, p["g1_W"], p["g1_as"], p["g1_ad"], p["g1_b"], 4, 32, True)
    h = jax.nn.relu(_bn(h, p["bn1_g"], p["bn1_b"], p["bn1_m"], p["bn1_v"]))
    h = _gat(h, src, dst, p["g2_W"], p["g2_as"], p["g2_ad"], p["g2_b"], 4, 32, True)
    h = jax.nn.relu(_bn(h, p["bn2_g"], p["bn2_b"], p["bn2_m"], p["bn2_v"]))
    h = _gat(h, src, dst, p["g3_W"], p["g3_as"], p["g3_ad"], p["g3_b"], 1, HID, False)
    h = jax.nn.relu(_bn(h, p["bn3_g"], p["bn3_b"], p["bn3_m"], p["bn3_v"]))
    pe = jax.nn.relu(platform_x @ p["pe_W"] + p["pe_b"])
    b1 = (pe @ p["c1_W"][HID:] + p["c1_b"])
    z = _mm(h, p["c1_W"][:HID], b1, relu=True)
    z = _mm(z, p["c2_W"], p["c2_b"][None, :], relu=True)
    out = z @ p["c3_W"] + p["c3_b"]
    return out
